# Initial kernel scaffold; baseline (speedup 1.0000x reference)
import jax
import jax.numpy as jnp
from jax.experimental import pallas as pl


def kernel(inputs, emb_weight):
    raise NotImplementedError("write your pallas kernel here")


# speedup vs baseline: 1.2780x; 1.2780x over previous
import jax
import jax.numpy as jnp
from jax.experimental import pallas as pl
from jax.experimental.pallas import tpu as pltpu

_B = 32
_C = 64
_HW = 4096
_K = 512
_N = _B * _HW


def _vq_block(x_ref, e_ref, q_ref, idx_ref, ssq_ref, cnt_ref):
    x = x_ref[0]
    emb = e_ref[...]
    xt = x.T
    xsq = jnp.sum(xt * xt, axis=1, keepdims=True)
    esq = jnp.sum(emb * emb, axis=1)
    mm = jax.lax.dot_general(xt, emb, (((1,), (1,)), ((), ())),
                             preferred_element_type=jnp.float32)
    d = (xsq + esq[None, :]) - 2.0 * mm
    m = jnp.min(d, axis=1, keepdims=True)
    lanes = jax.lax.broadcasted_iota(jnp.int32, (_HW, _K), 1)
    idx = jnp.min(jnp.where(d == m, lanes, _K), axis=1).astype(jnp.int32)
    idx_ref[0, 0, :] = idx
    oh = (jax.lax.broadcasted_iota(jnp.int32, (_K, _HW), 0)
          == idx[None, :]).astype(jnp.float32)
    q = jax.lax.dot_general(emb, oh, (((0,), (0,)), ((), ())),
                            preferred_element_type=jnp.float32)
    q_ref[0] = x + (q - x)
    ssq_ref[0, 0, :] = jnp.full((128,), jnp.sum((q - x) ** 2), jnp.float32)
    cnt_ref[0, 0, :] = jnp.sum(oh, axis=1)


def kernel(inputs, emb_weight):
    xr = inputs.reshape(_B, _C, _HW)
    q, idx, ssq, cnt = pl.pallas_call(
        _vq_block,
        grid=(_B,),
        in_specs=[
            pl.BlockSpec((1, _C, _HW), lambda b: (b, 0, 0)),
            pl.BlockSpec((_K, _C), lambda b: (0, 0)),
        ],
        out_specs=[
            pl.BlockSpec((1, _C, _HW), lambda b: (b, 0, 0)),
            pl.BlockSpec((1, 1, _HW), lambda b: (b, 0, 0)),
            pl.BlockSpec((1, 1, 128), lambda b: (b, 0, 0)),
            pl.BlockSpec((1, 1, _K), lambda b: (b, 0, 0)),
        ],
        out_shape=[
            jax.ShapeDtypeStruct((_B, _C, _HW), jnp.float32),
            jax.ShapeDtypeStruct((_B, 1, _HW), jnp.int32),
            jax.ShapeDtypeStruct((_B, 1, 128), jnp.float32),
            jax.ShapeDtypeStruct((_B, 1, _K), jnp.float32),
        ],
    )(xr, emb_weight)
    quantized = q.reshape(inputs.shape)
    indices = idx.reshape(_N)
    m = jnp.sum(ssq[:, 0, 0]) / (_N * _C)
    loss = m + 0.25 * m
    avg_probs = jnp.sum(cnt[:, 0, :], axis=0) / _N
    perplexity = jnp.exp(-jnp.sum(avg_probs * jnp.log(avg_probs + 1e-10)))
    return loss, quantized, perplexity, indices


# speedup vs baseline: 1.7410x; 1.3623x over previous
import jax
import jax.numpy as jnp
from jax.experimental import pallas as pl
from jax.experimental.pallas import tpu as pltpu

_B = 32
_C = 64
_HW = 4096
_K = 512
_N = _B * _HW


def _vq_block(x_ref, e_ref, q_ref, idx_ref, ssq_ref, cnt_ref, esq_ref):
    emb = e_ref[...]

    @pl.when(pl.program_id(0) == 0)
    def _init():
        esq_ref[...] = jnp.sum(emb * emb, axis=1, keepdims=True)

    x = x_ref[0]
    xsq = jnp.sum(x * x, axis=0, keepdims=True)
    mm = jax.lax.dot_general(emb, x, (((1,), (0,)), ((), ())),
                             preferred_element_type=jnp.float32)
    d = (xsq + esq_ref[...]) - 2.0 * mm
    m = jnp.min(d, axis=0, keepdims=True)
    codes = jax.lax.broadcasted_iota(jnp.int32, (_K, _HW), 0)
    idx = jnp.min(jnp.where(d == m, codes, _K), axis=0).astype(jnp.int32)
    idx_ref[0, 0, :] = idx
    oh = (codes == idx[None, :]).astype(jnp.float32)
    q = jax.lax.dot_general(emb, oh, (((0,), (0,)), ((), ())),
                            preferred_element_type=jnp.float32)
    err = q - x
    q_ref[0] = x + err
    ssq_ref[0, 0, :] = jnp.full((128,), jnp.sum(err * err), jnp.float32)
    cnt_ref[0, :, :] = jnp.sum(oh, axis=1, keepdims=True)


def kernel(inputs, emb_weight):
    xr = inputs.reshape(_B, _C, _HW)
    q, idx, ssq, cnt = pl.pallas_call(
        _vq_block,
        grid=(_B,),
        in_specs=[
            pl.BlockSpec((1, _C, _HW), lambda b: (b, 0, 0)),
            pl.BlockSpec((_K, _C), lambda b: (0, 0)),
        ],
        out_specs=[
            pl.BlockSpec((1, _C, _HW), lambda b: (b, 0, 0)),
            pl.BlockSpec((1, 1, _HW), lambda b: (b, 0, 0)),
            pl.BlockSpec((1, 1, 128), lambda b: (b, 0, 0)),
            pl.BlockSpec((1, _K, 1), lambda b: (b, 0, 0)),
        ],
        out_shape=[
            jax.ShapeDtypeStruct((_B, _C, _HW), jnp.float32),
            jax.ShapeDtypeStruct((_B, 1, _HW), jnp.int32),
            jax.ShapeDtypeStruct((_B, 1, 128), jnp.float32),
            jax.ShapeDtypeStruct((_B, _K, 1), jnp.float32),
        ],
        scratch_shapes=[pltpu.VMEM((_K, 1), jnp.float32)],
    )(xr, emb_weight)
    quantized = q.reshape(inputs.shape)
    indices = idx.reshape(_N)
    m = jnp.sum(ssq[:, 0, 0]) / (_N * _C)
    loss = m + 0.25 * m
    avg_probs = jnp.sum(cnt[:, :, 0], axis=0) / _N
    perplexity = jnp.exp(-jnp.sum(avg_probs * jnp.log(avg_probs + 1e-10)))
    return loss, quantized, perplexity, indices


# speedup vs baseline: 2.6637x; 1.5300x over previous
import jax
import jax.numpy as jnp
from jax.experimental import pallas as pl
from jax.experimental.pallas import tpu as pltpu

_B = 32
_C = 64
_HW = 4096
_K = 512
_N = _B * _HW


def _vq_block(x_ref, e_ref, q_ref, idx_ref, ssq_ref, cnt_ref, esq_ref):
    emb = e_ref[...]

    @pl.when(pl.program_id(0) == 0)
    def _init():
        esq_ref[...] = jnp.sum(emb * emb, axis=1, keepdims=True)

    x = x_ref[0].reshape(_C, _HW)
    xsq = jnp.sum(x * x, axis=0, keepdims=True)
    mm = jax.lax.dot_general(emb, x, (((1,), (0,)), ((), ())),
                             preferred_element_type=jnp.float32)
    d = (xsq + esq_ref[...]) - 2.0 * mm
    m = jnp.min(d, axis=0, keepdims=True)
    codes = jax.lax.broadcasted_iota(jnp.int32, (_K, _HW), 0)
    idx = jnp.min(jnp.where(d == m, codes, _K), axis=0).astype(jnp.int32)
    idx_ref[0, 0, :] = idx
    oh = (codes == idx[None, :]).astype(jnp.float32)
    q = jax.lax.dot_general(emb, oh, (((0,), (0,)), ((), ())),
                            preferred_element_type=jnp.float32)
    err = q - x
    q_ref[0] = (x + err).reshape(_C, 64, 64)
    ssq_ref[0, 0, :] = jnp.full((128,), jnp.sum(err * err), jnp.float32)
    cnt_ref[0, :, :] = jnp.sum(oh, axis=1, keepdims=True)


def kernel(inputs, emb_weight):
    quantized, idx, ssq, cnt = pl.pallas_call(
        _vq_block,
        grid=(_B,),
        in_specs=[
            pl.BlockSpec((1, _C, 64, 64), lambda b: (b, 0, 0, 0)),
            pl.BlockSpec((_K, _C), lambda b: (0, 0)),
        ],
        out_specs=[
            pl.BlockSpec((1, _C, 64, 64), lambda b: (b, 0, 0, 0)),
            pl.BlockSpec((1, 1, _HW), lambda b: (b, 0, 0)),
            pl.BlockSpec((1, 1, 128), lambda b: (b, 0, 0)),
            pl.BlockSpec((1, _K, 1), lambda b: (b, 0, 0)),
        ],
        out_shape=[
            jax.ShapeDtypeStruct((_B, _C, 64, 64), jnp.float32),
            jax.ShapeDtypeStruct((_B, 1, _HW), jnp.int32),
            jax.ShapeDtypeStruct((_B, 1, 128), jnp.float32),
            jax.ShapeDtypeStruct((_B, _K, 1), jnp.float32),
        ],
        scratch_shapes=[pltpu.VMEM((_K, 1), jnp.float32)],
    )(inputs, emb_weight)
    indices = idx.reshape(_N)
    m = jnp.sum(ssq[:, 0, 0]) / (_N * _C)
    loss = m + 0.25 * m
    avg_probs = jnp.sum(cnt[:, :, 0], axis=0) / _N
    perplexity = jnp.exp(-jnp.sum(avg_probs * jnp.log(avg_probs + 1e-10)))
    return loss, quantized, perplexity, indices


# speedup vs baseline: 2.7743x; 1.0415x over previous
import jax
import jax.numpy as jnp
from jax.experimental import pallas as pl
from jax.experimental.pallas import tpu as pltpu

_B = 32
_C = 64
_HW = 4096
_K = 512
_N = _B * _HW


def _vq_block(x_ref, e_ref, e2_ref, q_ref, idx_ref, ssq_ref, cnt_ref, esq_ref):
    emb = e_ref[...]

    @pl.when(pl.program_id(0) == 0)
    def _init():
        esq_ref[...] = jnp.sum(emb * emb, axis=1, keepdims=True)

    x = x_ref[0].reshape(_C, _HW)
    xsq = jnp.sum(x * x, axis=0, keepdims=True)
    mm = jax.lax.dot_general(e2_ref[...], x, (((1,), (0,)), ((), ())),
                             preferred_element_type=jnp.float32)
    d = (xsq + esq_ref[...]) + mm
    m = jnp.min(d, axis=0, keepdims=True)
    codes = jax.lax.broadcasted_iota(jnp.int32, (_K, _HW), 0)
    idx = jnp.min(jnp.where(d == m, codes, _K), axis=0).astype(jnp.int32)
    idx_ref[0, 0, :] = idx
    oh = (codes == idx[None, :]).astype(jnp.float32)
    q = jax.lax.dot_general(emb, oh, (((0,), (0,)), ((), ())),
                            preferred_element_type=jnp.float32)
    err = q - x
    q_ref[0] = (x + err).reshape(_C, 64, 64)
    ssq_ref[0, 0, :] = jnp.full((128,), jnp.sum(err * err), jnp.float32)
    cnt_ref[0, :, :] = jnp.sum(oh, axis=1, keepdims=True)


def kernel(inputs, emb_weight):
    quantized, idx, ssq, cnt = pl.pallas_call(
        _vq_block,
        grid=(_B,),
        in_specs=[
            pl.BlockSpec((1, _C, 64, 64), lambda b: (b, 0, 0, 0)),
            pl.BlockSpec((_K, _C), lambda b: (0, 0)),
            pl.BlockSpec((_K, _C), lambda b: (0, 0)),
        ],
        out_specs=[
            pl.BlockSpec((1, _C, 64, 64), lambda b: (b, 0, 0, 0)),
            pl.BlockSpec((1, 1, _HW), lambda b: (b, 0, 0)),
            pl.BlockSpec((1, 1, 128), lambda b: (b, 0, 0)),
            pl.BlockSpec((1, _K, 1), lambda b: (b, 0, 0)),
        ],
        out_shape=[
            jax.ShapeDtypeStruct((_B, _C, 64, 64), jnp.float32),
            jax.ShapeDtypeStruct((_B, 1, _HW), jnp.int32),
            jax.ShapeDtypeStruct((_B, 1, 128), jnp.float32),
            jax.ShapeDtypeStruct((_B, _K, 1), jnp.float32),
        ],
        scratch_shapes=[pltpu.VMEM((_K, 1), jnp.float32)],
    )(inputs, emb_weight, -2.0 * emb_weight)
    indices = idx.reshape(_N)
    m = jnp.sum(ssq[:, 0, 0]) / (_N * _C)
    loss = m + 0.25 * m
    avg_probs = jnp.sum(cnt[:, :, 0], axis=0) / _N
    perplexity = jnp.exp(-jnp.sum(avg_probs * jnp.log(avg_probs + 1e-10)))
    return loss, quantized, perplexity, indices
